# Initial kernel scaffold; baseline (speedup 1.0000x reference)
import jax
import jax.numpy as jnp
from jax.experimental import pallas as pl


def kernel(X):
    raise NotImplementedError("write your pallas kernel here")


# speedup vs baseline: 1.4421x; 1.4421x over previous
import jax
import jax.numpy as jnp
from jax.experimental import pallas as pl


_MAPS_PER_BLOCK = 16


def _wta_block(x_ref, o_ref):
    x = x_ref[...]
    m = jnp.max(x, axis=(1, 2), keepdims=True)
    o_ref[...] = jnp.where(x == m, x, jnp.zeros_like(x))


def kernel(X):
    B, C, H, W = X.shape
    N = B * C
    Xr = X.reshape(N, H, W)
    maps = _MAPS_PER_BLOCK
    if N % maps:
        maps = 1
    out = pl.pallas_call(
        _wta_block,
        grid=(N // maps,),
        in_specs=[pl.BlockSpec((maps, H, W), lambda i: (i, 0, 0))],
        out_specs=pl.BlockSpec((maps, H, W), lambda i: (i, 0, 0)),
        out_shape=jax.ShapeDtypeStruct((N, H, W), X.dtype),
    )(Xr)
    return out.reshape(B, C, H, W)


# speedup vs baseline: 1.4774x; 1.0245x over previous
import jax
import jax.numpy as jnp
from jax.experimental import pallas as pl
from jax.experimental.pallas import tpu as pltpu


_MAPS_PER_BLOCK = 16


def _wta_block(x_ref, o_ref):
    x = x_ref[...]
    m = jnp.max(x, axis=(1, 2), keepdims=True)
    o_ref[...] = jnp.where(x == m, x, jnp.zeros_like(x))


def kernel(X):
    B, C, H, W = X.shape
    N = B * C
    Xr = X.reshape(N, H, W)
    maps = _MAPS_PER_BLOCK
    if N % maps:
        maps = 1
    out = pl.pallas_call(
        _wta_block,
        grid=(N // maps,),
        in_specs=[pl.BlockSpec((maps, H, W), lambda i: (i, 0, 0))],
        out_specs=pl.BlockSpec((maps, H, W), lambda i: (i, 0, 0)),
        out_shape=jax.ShapeDtypeStruct((N, H, W), X.dtype),
        compiler_params=pltpu.CompilerParams(
            dimension_semantics=("parallel",),
        ),
    )(Xr)
    return out.reshape(B, C, H, W)


# speedup vs baseline: 1.4788x; 1.0009x over previous
import jax
import jax.numpy as jnp
from jax.experimental import pallas as pl
from jax.experimental.pallas import tpu as pltpu


_MAPS_PER_BLOCK = 32


def _wta_block(x_ref, o_ref):
    x = x_ref[...]
    m = jnp.max(x, axis=(1, 2), keepdims=True)
    o_ref[...] = jnp.where(x == m, x, jnp.zeros_like(x))


def kernel(X):
    B, C, H, W = X.shape
    N = B * C
    Xr = X.reshape(N, H, W)
    maps = _MAPS_PER_BLOCK
    if N % maps:
        maps = 1
    out = pl.pallas_call(
        _wta_block,
        grid=(N // maps,),
        in_specs=[pl.BlockSpec((maps, H, W), lambda i: (i, 0, 0))],
        out_specs=pl.BlockSpec((maps, H, W), lambda i: (i, 0, 0)),
        out_shape=jax.ShapeDtypeStruct((N, H, W), X.dtype),
        compiler_params=pltpu.CompilerParams(
            dimension_semantics=("parallel",),
        ),
    )(Xr)
    return out.reshape(B, C, H, W)


# speedup vs baseline: 1.4944x; 1.0105x over previous
import jax
import jax.numpy as jnp
from jax.experimental import pallas as pl
from jax.experimental.pallas import tpu as pltpu


_MAPS_PER_BLOCK = 64


def _wta_block(x_ref, o_ref):
    x = x_ref[...]
    m = jnp.max(x, axis=(1, 2), keepdims=True)
    o_ref[...] = jnp.where(x == m, x, jnp.zeros_like(x))


def kernel(X):
    B, C, H, W = X.shape
    N = B * C
    Xr = X.reshape(N, H, W)
    maps = _MAPS_PER_BLOCK
    if N % maps:
        maps = 1
    out = pl.pallas_call(
        _wta_block,
        grid=(N // maps,),
        in_specs=[pl.BlockSpec((maps, H, W), lambda i: (i, 0, 0))],
        out_specs=pl.BlockSpec((maps, H, W), lambda i: (i, 0, 0)),
        out_shape=jax.ShapeDtypeStruct((N, H, W), X.dtype),
        compiler_params=pltpu.CompilerParams(
            dimension_semantics=("parallel",),
        ),
    )(Xr)
    return out.reshape(B, C, H, W)
